# baseline (speedup 1.0000x reference)
import jax
import jax.numpy as jnp
from jax.experimental import pallas as pl
from jax.experimental.pallas import tpu as pltpu

_BLOCK_B = 2


def _noising_kernel(t_ref, betas_bar_ref, x_ref, noise_ref, out_ref, noise_out_ref):
    t = t_ref[0]
    beta = betas_bar_ref[t, 0]
    sa = jnp.sqrt(1.0 - beta)
    sb = jnp.sqrt(beta)
    n = noise_ref[...]
    out_ref[...] = sa * x_ref[...] + sb * n
    noise_out_ref[...] = n


def kernel(x, t, betas_bar, noise):
    t_arr = jnp.asarray(t, dtype=jnp.int32).reshape((1,))
    b, c, h, w = x.shape
    blk = (_BLOCK_B, c, h, w)
    noised, noise_out = pl.pallas_call(
        _noising_kernel,
        grid=(b // _BLOCK_B,),
        in_specs=[
            pl.BlockSpec(memory_space=pltpu.SMEM),
            pl.BlockSpec(memory_space=pltpu.SMEM),
            pl.BlockSpec(blk, lambda i: (i, 0, 0, 0)),
            pl.BlockSpec(blk, lambda i: (i, 0, 0, 0)),
        ],
        out_specs=[
            pl.BlockSpec(blk, lambda i: (i, 0, 0, 0)),
            pl.BlockSpec(blk, lambda i: (i, 0, 0, 0)),
        ],
        out_shape=[
            jax.ShapeDtypeStruct(x.shape, x.dtype),
            jax.ShapeDtypeStruct(x.shape, x.dtype),
        ],
    )(t_arr, betas_bar, x, noise)
    return noised, noise_out


# speedup vs baseline: 1.1192x; 1.1192x over previous
import jax
import jax.numpy as jnp
from jax.experimental import pallas as pl
from jax.experimental.pallas import tpu as pltpu

_BLOCK_B = 8


def _noising_kernel(t_ref, betas_bar_ref, x_ref, noise_ref, out_ref, noise_out_ref):
    t = t_ref[0]
    beta = betas_bar_ref[t, 0]
    sa = jnp.sqrt(1.0 - beta)
    sb = jnp.sqrt(beta)
    n = noise_ref[...]
    out_ref[...] = sa * x_ref[...] + sb * n
    noise_out_ref[...] = n


def kernel(x, t, betas_bar, noise):
    t_arr = jnp.asarray(t, dtype=jnp.int32).reshape((1,))
    b, c, h, w = x.shape
    blk = (_BLOCK_B, c, h, w)
    noised, noise_out = pl.pallas_call(
        _noising_kernel,
        grid=(b // _BLOCK_B,),
        in_specs=[
            pl.BlockSpec(memory_space=pltpu.SMEM),
            pl.BlockSpec(memory_space=pltpu.SMEM),
            pl.BlockSpec(blk, lambda i: (i, 0, 0, 0)),
            pl.BlockSpec(blk, lambda i: (i, 0, 0, 0)),
        ],
        out_specs=[
            pl.BlockSpec(blk, lambda i: (i, 0, 0, 0)),
            pl.BlockSpec(blk, lambda i: (i, 0, 0, 0)),
        ],
        out_shape=[
            jax.ShapeDtypeStruct(x.shape, x.dtype),
            jax.ShapeDtypeStruct(x.shape, x.dtype),
        ],
    )(t_arr, betas_bar, x, noise)
    return noised, noise_out
